# Initial kernel scaffold; baseline (speedup 1.0000x reference)
import jax
import jax.numpy as jnp
from jax.experimental import pallas as pl


def kernel(x, codebook):
    raise NotImplementedError("write your pallas kernel here")


# speedup vs baseline: 1.5261x; 1.5261x over previous
import functools

import jax
import jax.numpy as jnp
from jax import lax
from jax.experimental import pallas as pl
from jax.experimental.pallas import tpu as pltpu
from jax.experimental.pallas import tpu_sc as plsc


def _dist_argmin_body(x_ref, cb_ref, z_ref, acc_ref):
    xb = x_ref[...]
    cb = cb_ref[...]
    ab = lax.dot_general(xb, cb, (((1,), (1,)), ((), ())),
                         preferred_element_type=jnp.float32)
    a2 = jnp.sum(xb * xb, axis=1, keepdims=True)
    b2 = jnp.sum(cb * cb, axis=1)[None, :]
    d = (a2 + b2) - 2.0 * ab
    minv = jnp.min(d, axis=1, keepdims=True)
    k = d.shape[1]
    idx = lax.broadcasted_iota(jnp.int32, d.shape, 1)
    z = jnp.min(jnp.where(d == minv, idx, k), axis=1, keepdims=True)
    z_ref[...] = z

    i = pl.program_id(0)

    @pl.when(i == 0)
    def _init():
        acc_ref[0, 0] = 0.0

    acc_ref[0, 0] += jnp.sum(minv)


def _dist_argmin(x, codebook, block_rows):
    n, d = x.shape
    k = codebook.shape[0]
    return pl.pallas_call(
        _dist_argmin_body,
        grid=(n // block_rows,),
        in_specs=[
            pl.BlockSpec((block_rows, d), lambda i: (i, 0)),
            pl.BlockSpec((k, d), lambda i: (0, 0)),
        ],
        out_specs=[
            pl.BlockSpec((block_rows, 1), lambda i: (i, 0)),
            pl.BlockSpec(memory_space=pltpu.SMEM),
        ],
        out_shape=[
            jax.ShapeDtypeStruct((n, 1), jnp.int32),
            jax.ShapeDtypeStruct((1, 1), jnp.float32),
        ],
    )(x, codebook)


@functools.cache
def _make_sc_gather(v, d, b, dtype):
    info = plsc.get_sparse_core_info()
    nc, ns = info.num_cores, info.num_subcores
    nw = nc * ns
    b_per_w = b // nw
    mesh = plsc.VectorSubcoreMesh(core_axis_name="c", subcore_axis_name="s")

    @functools.partial(
        pl.kernel, mesh=mesh,
        compiler_params=pltpu.CompilerParams(use_tc_tiling_on_sc=False),
        out_type=jax.ShapeDtypeStruct((b, d), dtype),
        scratch_types=[
            pltpu.VMEM((b_per_w,), jnp.int32),
            pltpu.VMEM((b_per_w, d), dtype),
            pltpu.SemaphoreType.DMA,
        ],
    )
    def gather(table_hbm, idx_hbm, out_hbm, idx_v, rows_v, sem):
        wid = lax.axis_index("s") * nc + lax.axis_index("c")
        base = wid * b_per_w
        pltpu.sync_copy(idx_hbm.at[pl.ds(base, b_per_w)], idx_v)
        pltpu.async_copy(table_hbm.at[idx_v], rows_v, sem).wait()
        pltpu.sync_copy(rows_v, out_hbm.at[pl.ds(base, b_per_w)])

    return gather


def kernel(x, codebook):
    n, d = x.shape
    k = codebook.shape[0]
    z2, acc = _dist_argmin(x, codebook, 512)
    z = z2.reshape(n)
    q = _make_sc_gather(k, d, n, codebook.dtype)(codebook, z)
    loss = acc[0, 0] / jnp.float32(n * d)
    return (z, q, loss, loss)
